# baseline (speedup 1.0000x reference)
import functools
import math

import jax
import jax.numpy as jnp
from jax import lax
from jax.experimental import pallas as pl
from jax.experimental.pallas import tpu as pltpu
from jax.experimental.pallas import tpu_sc as plsc

_INFO = plsc.get_sparse_core_info()
_NC = _INFO.num_cores
_NS = _INFO.num_subcores
_L = _INFO.num_lanes
_NW = _NC * _NS

_CH = 320


@functools.lru_cache(maxsize=None)
def _make_call(B, V, D, scale):
    assert B % (_NW * _CH) == 0
    b_per_w = B // _NW
    n_chunks = b_per_w // _CH
    assert n_chunks % 2 == 0
    mesh = plsc.VectorSubcoreMesh(core_axis_name="c", subcore_axis_name="s")

    @functools.partial(
        pl.kernel,
        mesh=mesh,
        out_type=jax.ShapeDtypeStruct((B, 128), jnp.float32),
        scratch_types=[
            pltpu.VMEM((b_per_w,), jnp.int32),
            pltpu.VMEM((_CH, 128), jnp.float32),
            pltpu.VMEM((_CH, 128), jnp.float32),
            pltpu.SemaphoreType.DMA,
            pltpu.SemaphoreType.DMA,
        ],
        compiler_params=pltpu.CompilerParams(use_tc_tiling_on_sc=False),
    )
    def emb(idx_hbm, table_hbm, out_hbm, idx_v, rows0, rows1, sem0, sem1):
        wid = lax.axis_index("s") * _NC + lax.axis_index("c")
        base = wid * b_per_w
        bufs = (rows0, rows1)
        sems = (sem0, sem1)

        pltpu.sync_copy(idx_hbm.at[pl.ds(base, b_per_w)], idx_v)

        def g_start(c, b):
            pltpu.async_copy(
                table_hbm.at[idx_v.at[pl.ds(c * _CH, _CH)]], bufs[b], sems[b])

        def g_wait(b):
            pltpu.make_async_copy(
                table_hbm.at[idx_v.at[pl.ds(0, _CH)]], bufs[b], sems[b]).wait()

        def scale_rows(b):
            rows = bufs[b]

            def row_body(j, carry):
                for k in range(D // _L):
                    sl = pl.ds(k * _L, _L)
                    rows[j, sl] = rows[j, sl] * scale
                return carry

            lax.fori_loop(0, _CH, row_body, 0, unroll=8)

        g_start(0, 0)

        def pair_body(i, carry):
            for j in range(2):
                c = i * 2 + j
                b = j

                @pl.when(c + 1 < n_chunks)
                def _():
                    g_start(c + 1, 1 - b)

                g_wait(b)
                scale_rows(b)
                pltpu.sync_copy(
                    bufs[b].at[:, pl.ds(0, D)],
                    out_hbm.at[pl.ds(base + c * _CH, _CH), pl.ds(0, D)])
            return carry

        lax.fori_loop(0, n_chunks // 2, pair_body, 0)

    return emb


def kernel(x, table):
    Bdim, S = x.shape
    V, D = table.shape
    idx = x.reshape(-1).astype(jnp.int32)
    scale = float(math.sqrt(D))
    tbl_pad = lax.optimization_barrier(jnp.pad(table, ((0, 0), (0, 128 - D))))
    out = _make_call(Bdim * S, V, D, scale)(idx, tbl_pad)
    return out.reshape(Bdim, S, 128)[:, :, :D]


# speedup vs baseline: 1.0106x; 1.0106x over previous
import functools
import math

import jax
import jax.numpy as jnp
from jax import lax
from jax.experimental import pallas as pl
from jax.experimental.pallas import tpu as pltpu
from jax.experimental.pallas import tpu_sc as plsc

_INFO = plsc.get_sparse_core_info()
_NC = _INFO.num_cores
_NS = _INFO.num_subcores
_L = _INFO.num_lanes
_NW = _NC * _NS

_CH = 640


@functools.lru_cache(maxsize=None)
def _make_call(B, V, D, scale):
    assert B % (_NW * _CH) == 0
    b_per_w = B // _NW
    n_chunks = b_per_w // _CH
    assert n_chunks % 2 == 0
    mesh = plsc.VectorSubcoreMesh(core_axis_name="c", subcore_axis_name="s")

    @functools.partial(
        pl.kernel,
        mesh=mesh,
        out_type=jax.ShapeDtypeStruct((B, 128), jnp.float32),
        scratch_types=[
            pltpu.VMEM((b_per_w,), jnp.int32),
            pltpu.VMEM((_CH, D), jnp.float32),
            pltpu.VMEM((_CH, D), jnp.float32),
            pltpu.SemaphoreType.DMA,
            pltpu.SemaphoreType.DMA,
        ],
        compiler_params=pltpu.CompilerParams(use_tc_tiling_on_sc=False),
    )
    def emb(idx_hbm, table_hbm, out_hbm, idx_v, rows0, rows1, sem0, sem1):
        wid = lax.axis_index("s") * _NC + lax.axis_index("c")
        base = wid * b_per_w
        bufs = (rows0, rows1)
        sems = (sem0, sem1)

        pltpu.sync_copy(idx_hbm.at[pl.ds(base, b_per_w)], idx_v)

        def g_start(c, b):
            pltpu.async_copy(
                table_hbm.at[idx_v.at[pl.ds(c * _CH, _CH)]], bufs[b], sems[b])

        def g_wait(b):
            pltpu.make_async_copy(
                table_hbm.at[idx_v.at[pl.ds(0, _CH)]], bufs[b], sems[b]).wait()

        def scale_rows(b):
            rows = bufs[b]

            def row_body(j, carry):
                for k in range(D // _L):
                    sl = pl.ds(k * _L, _L)
                    rows[j, sl] = rows[j, sl] * scale
                return carry

            lax.fori_loop(0, _CH, row_body, 0, unroll=8)

        g_start(0, 0)

        def pair_body(i, carry):
            for j in range(2):
                c = i * 2 + j
                b = j

                @pl.when(c + 1 < n_chunks)
                def _():
                    g_start(c + 1, 1 - b)

                g_wait(b)
                scale_rows(b)
                pltpu.sync_copy(
                    bufs[b],
                    out_hbm.at[pl.ds(base + c * _CH, _CH), pl.ds(0, D)])
            return carry

        lax.fori_loop(0, n_chunks // 2, pair_body, 0)

    return emb


def kernel(x, table):
    Bdim, S = x.shape
    V, D = table.shape
    idx = x.reshape(-1).astype(jnp.int32)
    scale = float(math.sqrt(D))
    tbl2 = lax.optimization_barrier(table.reshape(V * D // 128, 128))
    tbl_lin = tbl2.reshape(V, D)
    out = _make_call(Bdim * S, V, D, scale)(idx, tbl_lin)
    return out.reshape(Bdim, S, 128)[:, :, :D]


# speedup vs baseline: 1.0133x; 1.0027x over previous
import functools
import math

import jax
import jax.numpy as jnp
from jax import lax
from jax.experimental import pallas as pl
from jax.experimental.pallas import tpu as pltpu
from jax.experimental.pallas import tpu_sc as plsc

_INFO = plsc.get_sparse_core_info()
_NC = _INFO.num_cores
_NS = _INFO.num_subcores
_L = _INFO.num_lanes
_NW = _NC * _NS

_CH = 512
_NBUF = 3


@functools.lru_cache(maxsize=None)
def _make_call(B, V, D, scale):
    assert B % (_NW * _CH) == 0
    b_per_w = B // _NW
    n_chunks = b_per_w // _CH
    n_outer = (n_chunks + _NBUF - 1) // _NBUF
    mesh = plsc.VectorSubcoreMesh(core_axis_name="c", subcore_axis_name="s")

    @functools.partial(
        pl.kernel,
        mesh=mesh,
        out_type=jax.ShapeDtypeStruct((B, 128), jnp.float32),
        scratch_types=[
            pltpu.VMEM((b_per_w,), jnp.int32),
            pltpu.VMEM((_CH, D), jnp.float32),
            pltpu.VMEM((_CH, D), jnp.float32),
            pltpu.VMEM((_CH, D), jnp.float32),
            pltpu.SemaphoreType.DMA,
            pltpu.SemaphoreType.DMA,
            pltpu.SemaphoreType.DMA,
            pltpu.SemaphoreType.DMA,
            pltpu.SemaphoreType.DMA,
            pltpu.SemaphoreType.DMA,
        ],
        compiler_params=pltpu.CompilerParams(use_tc_tiling_on_sc=False),
    )
    def emb(idx_hbm, table_hbm, out_hbm, idx_v, r0, r1, r2,
            gs0, gs1, gs2, os0, os1, os2):
        wid = lax.axis_index("s") * _NC + lax.axis_index("c")
        base = wid * b_per_w
        bufs = (r0, r1, r2)
        gsems = (gs0, gs1, gs2)
        osems = (os0, os1, os2)

        pltpu.sync_copy(idx_hbm.at[pl.ds(base, b_per_w)], idx_v)

        def g_start(c, b):
            pltpu.async_copy(
                table_hbm.at[idx_v.at[pl.ds(c * _CH, _CH)]], bufs[b],
                gsems[b])

        def g_wait(b):
            pltpu.make_async_copy(
                table_hbm.at[idx_v.at[pl.ds(0, _CH)]], bufs[b],
                gsems[b]).wait()

        def o_start(c, b):
            pltpu.async_copy(
                bufs[b],
                out_hbm.at[pl.ds(base + c * _CH, _CH), pl.ds(0, D)],
                osems[b])

        def o_wait(b):
            pltpu.make_async_copy(
                bufs[b],
                out_hbm.at[pl.ds(base, _CH), pl.ds(0, D)],
                osems[b]).wait()

        def scale_rows(b):
            rows = bufs[b]

            def row_body(j, carry):
                for k in range(D // _L):
                    sl = pl.ds(k * _L, _L)
                    rows[j, sl] = rows[j, sl] * scale
                return carry

            lax.fori_loop(0, _CH, row_body, 0, unroll=8)

        g_start(0, 0)

        def outer_body(i, carry):
            for j in range(_NBUF):
                c = i * _NBUF + j
                b = j

                @pl.when(c < n_chunks)
                def _():
                    nb = (j + 1) % _NBUF

                    @pl.when(c + 1 < n_chunks)
                    def _():
                        @pl.when(c + 1 >= _NBUF)
                        def _():
                            o_wait(nb)

                        g_start(c + 1, nb)

                    g_wait(b)
                    scale_rows(b)
                    o_start(c, b)
            return carry

        lax.fori_loop(0, n_outer, outer_body, 0)

        for j in range(_NBUF):
            c = n_chunks - _NBUF + j
            if c >= 0:
                o_wait(c % _NBUF)

    return emb


def kernel(x, table):
    Bdim, S = x.shape
    V, D = table.shape
    idx = x.reshape(-1).astype(jnp.int32)
    scale = float(math.sqrt(D))
    tbl2 = lax.optimization_barrier(table.reshape(V * D // 128, 128))
    tbl_lin = tbl2.reshape(V, D)
    out = _make_call(Bdim * S, V, D, scale)(idx, tbl_lin)
    return out.reshape(Bdim, S, 128)[:, :, :D]
